# speedup vs baseline: 1.0547x; 1.0547x over previous
import functools

import jax
import jax.numpy as jnp
from jax import lax
from jax.experimental import pallas as pl
from jax.experimental.pallas import tpu as pltpu

N_NODE = 50000; N_NET = 10000; E_PIN = 100000; E_GRID = 400000
L = 2; H = 2; GO = 16; C = 2
NODE_F = 16; NET_F = 32; HID_NODE = NODE_F + GO * C * H
PIN_F = 16; IN_NODE = 128; IN_NET = 64; IN_PIN = 16; NT = 4


def _leaky(x, s):
    return jnp.where(x >= 0, x, s * x)


def _mm_body(x_ref, w_ref, b_ref, o_ref, *, act):
    y = jnp.dot(x_ref[...], w_ref[...], preferred_element_type=jnp.float32)
    y = y + b_ref[...]
    if act == "leaky":
        y = _leaky(y, 0.01)
    elif act == "tanh":
        y = jnp.tanh(y)
    o_ref[...] = y


def _mm(x, w, b, act="none", tile=2000):
    m, k = x.shape
    n = w.shape[1]
    assert m % tile == 0, (m, tile)
    return pl.pallas_call(
        functools.partial(_mm_body, act=act),
        grid=(m // tile,),
        in_specs=[
            pl.BlockSpec((tile, k), lambda i: (i, 0)),
            pl.BlockSpec((k, n), lambda i: (0, 0)),
            pl.BlockSpec((1, n), lambda i: (0, 0)),
        ],
        out_specs=pl.BlockSpec((tile, n), lambda i: (i, 0)),
        out_shape=jax.ShapeDtypeStruct((m, n), jnp.float32),
    )(x, w, b.reshape(1, n))


def _mlp_body(x_ref, w1_ref, b1_ref, w2_ref, b2_ref, w3_ref, b3_ref, o_ref):
    h = jnp.tanh(jnp.dot(x_ref[...], w1_ref[...],
                         preferred_element_type=jnp.float32) + b1_ref[...])
    h = jnp.tanh(jnp.dot(h, w2_ref[...],
                         preferred_element_type=jnp.float32) + b2_ref[...])
    y = jnp.dot(h, w3_ref[...], preferred_element_type=jnp.float32) + b3_ref[...]
    o_ref[...] = jax.nn.sigmoid(y)


def _mlp(x, w1, b1, w2, b2, w3, b3, tile=2000):
    m, k = x.shape
    h1 = w1.shape[1]
    h2 = w2.shape[1]
    n = w3.shape[1]
    return pl.pallas_call(
        _mlp_body,
        grid=(m // tile,),
        in_specs=[
            pl.BlockSpec((tile, k), lambda i: (i, 0)),
            pl.BlockSpec((k, h1), lambda i: (0, 0)),
            pl.BlockSpec((1, h1), lambda i: (0, 0)),
            pl.BlockSpec((h1, h2), lambda i: (0, 0)),
            pl.BlockSpec((1, h2), lambda i: (0, 0)),
            pl.BlockSpec((h2, n), lambda i: (0, 0)),
            pl.BlockSpec((1, n), lambda i: (0, 0)),
        ],
        out_specs=pl.BlockSpec((tile, n), lambda i: (i, 0)),
        out_shape=jax.ShapeDtypeStruct((m, n), jnp.float32),
    )(x, w1, b1.reshape(1, h1), w2, b2.reshape(1, h2), w3, b3.reshape(1, n))


def _seg_sum(vals, idx, num):
    return jax.ops.segment_sum(vals, idx, num_segments=num)


def _gat_layer(node, gat_W, al, ar, gat_b, grid_edges):
    h = _mm(node, gat_W, jnp.zeros((H * GO,), jnp.float32))
    Wl = (gat_W.reshape(HID_NODE, H, GO) * al[None]).sum(-1)
    Wr = (gat_W.reshape(HID_NODE, H, GO) * ar[None]).sum(-1)
    el = _mm(node, Wl, jnp.zeros((H,), jnp.float32))
    er = _mm(node, Wr, jnp.zeros((H,), jnp.float32))
    gmax = jnp.max(el, axis=0)
    c = _leaky(er + gmax[None, :], 0.2)
    outs = []
    for j in range(C):
        src = grid_edges[j, 0]
        dst = grid_edges[j, 1]
        e = _leaky(el[src] + er[dst], 0.2)
        ee = jnp.exp(e - c[dst])
        den = _seg_sum(ee, dst, N_NODE)
        num = _seg_sum(h[src].reshape(-1, H, GO) * ee[:, :, None], dst, N_NODE)
        out = num / jnp.maximum(den, 1e-9)[:, :, None]
        outs.append(out + gat_b.reshape(1, H, GO))
    return jnp.concatenate(outs, axis=-1).reshape(N_NODE, GO * C * H)


def kernel(in_node_feat, in_net_feat, in_pin_feat,
           node_lin_W, node_lin_b, net_lin_W, net_lin_b, pin_lin_W, pin_lin_b,
           gat_W, gat_al, gat_ar, gat_b, lin2_W, lin2_b, gc_W, gc_b, nnc_b,
           out1_W, out1_b, out2_W, out2_b, out3_W, out3_b,
           pins_src, pins_dst, pinned_src, pinned_dst, grid_edges):
    node = _mm(in_node_feat, node_lin_W, node_lin_b, act="leaky")
    net = _mm(in_net_feat, net_lin_W, net_lin_b, act="leaky")
    pin = _mm(in_pin_feat, pin_lin_W, pin_lin_b, act="leaky")

    ones_pin = jnp.ones((E_PIN,), jnp.float32)
    dout = jnp.maximum(_seg_sum(ones_pin, pins_src, N_NODE), 1.0) ** -0.5
    din = jnp.maximum(_seg_sum(ones_pin, pins_dst, N_NET), 1.0) ** -0.5
    cnt = jnp.maximum(_seg_sum(ones_pin, pinned_dst, N_NODE), 1.0)

    W2r = [lin2_W[i].reshape(PIN_F, NET_F, NODE_F).transpose(1, 0, 2)
           .reshape(NET_F, PIN_F * NODE_F) for i in range(L)]
    b2r = [lin2_b[i].reshape(NET_F, NODE_F) for i in range(L)]

    for i in range(L):
        og = _gat_layer(node, gat_W[i], gat_al[i], gat_ar[i], gat_b[i],
                        grid_edges)
        T = _mm(net, jnp.concatenate([W2r[i], b2r[i]], axis=1),
                jnp.zeros((PIN_F * NODE_F + NODE_F,), jnp.float32),
                tile=2000)
        Tg = T[pinned_src]
        msg = (pin[:, :, None] * Tg[:, :PIN_F * NODE_F]
               .reshape(E_PIN, PIN_F, NODE_F)).sum(1) + Tg[:, PIN_F * NODE_F:]
        s = _seg_sum(msg, pinned_dst, N_NODE)
        on = s / cnt[:, None] + nnc_b[i]
        xw = _mm(node * dout[:, None], gc_W[i],
                 jnp.zeros((NET_F,), jnp.float32))
        agg = _seg_sum(xw[pins_src], pins_dst, N_NET)
        net = jnp.tanh(agg * din[:, None] + gc_b[i])
        node = jnp.tanh(jnp.concatenate([og, on], axis=-1))

    x = jnp.concatenate([in_node_feat, node], axis=-1)
    return _mlp(x, out1_W, out1_b, out2_W, out2_b, out3_W, out3_b)
